# baseline (speedup 1.0000x reference)
import functools

import jax
import jax.numpy as jnp
from jax import lax
from jax.experimental import pallas as pl
from jax.experimental.pallas import tpu as pltpu
from jax.experimental.pallas import tpu_sc as plsc

NUM = 16384
C = 1000
C_PAD = 1024

NC = 2
NS = 16
NW = NC * NS
BPW = NUM // NW
CHUNK = 128
NCH = BPW // CHUNK

ROWS_BLK = 4096
GRID = NUM // ROWS_BLK
W_ROWS = NUM // 128


def _sc_alpha_gather(tgt2d, alpha_pad):
    mesh = plsc.VectorSubcoreMesh(core_axis_name="c", subcore_axis_name="s")

    @functools.partial(
        pl.kernel,
        mesh=mesh,
        out_type=jax.ShapeDtypeStruct((W_ROWS, 128), jnp.float32),
        scratch_types=[
            pltpu.VMEM((NCH, CHUNK), jnp.int32),
            pltpu.VMEM((NCH, CHUNK), jnp.float32),
            pltpu.SemaphoreType.DMA,
        ],
    )
    def k(tgt_hbm, alpha_hbm, a_hbm, tgt_v, a_v, sem):
        wid = lax.axis_index("s") * NC + lax.axis_index("c")
        pltpu.sync_copy(tgt_hbm.at[pl.ds(NCH * wid, NCH)], tgt_v)
        copies = [
            pltpu.async_copy(alpha_hbm.at[tgt_v.at[ch]], a_v.at[ch], sem)
            for ch in range(NCH)
        ]
        for cp in copies:
            cp.wait()
        pltpu.sync_copy(a_v, a_hbm.at[pl.ds(NCH * wid, NCH)])

    return k(tgt2d, alpha_pad)


def _tc_main_body(t_ref, x_ref, w_ref):
    x = x_ref[...]
    t = t_ref[0, 0, :]
    rows = jax.lax.broadcasted_iota(jnp.int32, (C, ROWS_BLK), 0)
    mask = (rows == t[None, :]).astype(jnp.float32)
    p = jnp.sum(x * mask, axis=0)
    om = 1.0 - p
    w = (om * om) * jnp.log(p)
    w_ref[...] = w.reshape(ROWS_BLK // 128, 128)


def _tc_main(tgt, inputs):
    x_t = inputs.T
    t3d = tgt.reshape(GRID, 1, ROWS_BLK)
    return pl.pallas_call(
        _tc_main_body,
        grid=(GRID,),
        in_specs=[
            pl.BlockSpec((1, 1, ROWS_BLK), lambda i: (i, 0, 0)),
            pl.BlockSpec((C, ROWS_BLK), lambda i: (0, i)),
        ],
        out_specs=pl.BlockSpec((ROWS_BLK // 128, 128), lambda i: (i, 0)),
        out_shape=jax.ShapeDtypeStruct((W_ROWS, 128), jnp.float32),
    )(t3d, x_t)


def _tc_fin_body(a_ref, w_ref, o_ref):
    o_ref[0, 0] = jnp.sum(a_ref[...] * w_ref[...]) * (-1.0 / NUM)


def _tc_finish(a2d, w2d):
    out = pl.pallas_call(
        _tc_fin_body,
        out_shape=jax.ShapeDtypeStruct((1, 1), jnp.float32),
        out_specs=pl.BlockSpec(memory_space=pltpu.SMEM),
    )(a2d, w2d)
    return out[0, 0]


def kernel(inputs, targets, alpha):
    tgt = targets.astype(jnp.int32)
    tgt2d = tgt.reshape(W_ROWS, 128)
    al = jnp.pad(alpha.reshape(-1), (0, C_PAD - C))
    a2d = _sc_alpha_gather(tgt2d, al)
    w2d = _tc_main(tgt, inputs)
    return _tc_finish(a2d, w2d)


# speedup vs baseline: 1.0010x; 1.0010x over previous
import functools

import jax
import jax.numpy as jnp
from jax import lax
from jax.experimental import pallas as pl
from jax.experimental.pallas import tpu as pltpu
from jax.experimental.pallas import tpu_sc as plsc

NUM = 16384
C = 1000
C_PAD = 1024

NC = 2
NS = 16
NW = NC * NS
BPW = NUM // NW
CHUNK = 128
NCH = BPW // CHUNK

ROWS_BLK = 2048
GRID = NUM // ROWS_BLK
W_ROWS = NUM // 128


def _sc_alpha_gather(tgt2d, alpha_pad):
    mesh = plsc.VectorSubcoreMesh(core_axis_name="c", subcore_axis_name="s")

    @functools.partial(
        pl.kernel,
        mesh=mesh,
        out_type=jax.ShapeDtypeStruct((W_ROWS, 128), jnp.float32),
        scratch_types=[
            pltpu.VMEM((NCH, CHUNK), jnp.int32),
            pltpu.VMEM((NCH, CHUNK), jnp.float32),
            pltpu.SemaphoreType.DMA,
        ],
    )
    def k(tgt_hbm, alpha_hbm, a_hbm, tgt_v, a_v, sem):
        wid = lax.axis_index("s") * NC + lax.axis_index("c")
        pltpu.sync_copy(tgt_hbm.at[pl.ds(NCH * wid, NCH)], tgt_v)
        copies = [
            pltpu.async_copy(alpha_hbm.at[tgt_v.at[ch]], a_v.at[ch], sem)
            for ch in range(NCH)
        ]
        for cp in copies:
            cp.wait()
        pltpu.sync_copy(a_v, a_hbm.at[pl.ds(NCH * wid, NCH)])

    return k(tgt2d, alpha_pad)


def _tc_main_body(t_ref, x_ref, w_ref):
    x = x_ref[...]
    t = t_ref[0, 0, :]
    rows = jax.lax.broadcasted_iota(jnp.int32, (C, ROWS_BLK), 0)
    mask = (rows == t[None, :]).astype(jnp.float32)
    p = jnp.sum(x * mask, axis=0)
    om = 1.0 - p
    w = (om * om) * jnp.log(p)
    w_ref[...] = w.reshape(ROWS_BLK // 128, 128)


def _tc_main(tgt, inputs):
    x_t = inputs.T
    t3d = tgt.reshape(GRID, 1, ROWS_BLK)
    return pl.pallas_call(
        _tc_main_body,
        grid=(GRID,),
        in_specs=[
            pl.BlockSpec((1, 1, ROWS_BLK), lambda i: (i, 0, 0)),
            pl.BlockSpec((C, ROWS_BLK), lambda i: (0, i)),
        ],
        out_specs=pl.BlockSpec((ROWS_BLK // 128, 128), lambda i: (i, 0)),
        out_shape=jax.ShapeDtypeStruct((W_ROWS, 128), jnp.float32),
    )(t3d, x_t)


def _tc_fin_body(a_ref, w_ref, o_ref):
    o_ref[0, 0] = jnp.sum(a_ref[...] * w_ref[...]) * (-1.0 / NUM)


def _tc_finish(a2d, w2d):
    out = pl.pallas_call(
        _tc_fin_body,
        out_shape=jax.ShapeDtypeStruct((1, 1), jnp.float32),
        out_specs=pl.BlockSpec(memory_space=pltpu.SMEM),
    )(a2d, w2d)
    return out[0, 0]


def kernel(inputs, targets, alpha):
    tgt = targets.astype(jnp.int32)
    tgt2d = tgt.reshape(W_ROWS, 128)
    al = jnp.pad(alpha.reshape(-1), (0, C_PAD - C))
    a2d = _sc_alpha_gather(tgt2d, al)
    w2d = _tc_main(tgt, inputs)
    return _tc_finish(a2d, w2d)


# speedup vs baseline: 1.0126x; 1.0116x over previous
import functools

import jax
import jax.numpy as jnp
from jax import lax
from jax.experimental import pallas as pl
from jax.experimental.pallas import tpu as pltpu
from jax.experimental.pallas import tpu_sc as plsc

NUM = 16384
C = 1000
C_PAD = 1024

NC = 2
NS = 16
NW = NC * NS
BPW = NUM // NW
CHUNK = 128
NCH = BPW // CHUNK

ROWS_BLK = 2048
GRID = NUM // ROWS_BLK
W_ROWS = NUM // 128


def _sc_alpha_gather(tgt2d, alpha_pad):
    mesh = plsc.VectorSubcoreMesh(core_axis_name="c", subcore_axis_name="s")

    @functools.partial(
        pl.kernel,
        mesh=mesh,
        out_type=jax.ShapeDtypeStruct((W_ROWS, 128), jnp.float32),
        scratch_types=[
            pltpu.VMEM((NCH, CHUNK), jnp.int32),
            pltpu.VMEM((NCH, CHUNK), jnp.float32),
            pltpu.SemaphoreType.DMA,
        ],
    )
    def k(tgt_hbm, alpha_hbm, a_hbm, tgt_v, a_v, sem):
        wid = lax.axis_index("s") * NC + lax.axis_index("c")
        pltpu.sync_copy(tgt_hbm.at[pl.ds(NCH * wid, NCH)], tgt_v)
        copies = [
            pltpu.async_copy(alpha_hbm.at[tgt_v.at[ch]], a_v.at[ch], sem)
            for ch in range(NCH)
        ]
        for cp in copies:
            cp.wait()
        pltpu.sync_copy(a_v, a_hbm.at[pl.ds(NCH * wid, NCH)])

    return k(tgt2d, alpha_pad)


def _tc_main_body(t_ref, x0_ref, x1_ref, w0_ref, w1_ref):
    rows = jax.lax.broadcasted_iota(jnp.int32, (C, ROWS_BLK), 0)
    for x_ref, w_ref, half in ((x0_ref, w0_ref, 0), (x1_ref, w1_ref, 1)):
        th = t_ref[0, half, 0, :]
        mask = (rows == th[None, :]).astype(jnp.float32)
        p = jnp.sum(x_ref[...] * mask, axis=0)
        om = 1.0 - p
        w = (om * om) * jnp.log(p)
        w_ref[...] = w.reshape(ROWS_BLK // 128, 128)


def _tc_main(tgt, inputs):
    x_t = inputs.T
    half_steps = NUM // 2 // ROWS_BLK
    t3d = tgt.reshape(half_steps, 2, 1, ROWS_BLK)
    return pl.pallas_call(
        _tc_main_body,
        grid=(half_steps,),
        in_specs=[
            pl.BlockSpec((1, 2, 1, ROWS_BLK), lambda i: (i, 0, 0, 0)),
            pl.BlockSpec((C, ROWS_BLK), lambda i: (0, 2 * i)),
            pl.BlockSpec((C, ROWS_BLK), lambda i: (0, 2 * i + 1)),
        ],
        out_specs=[
            pl.BlockSpec((ROWS_BLK // 128, 128), lambda i: (i, 0)),
            pl.BlockSpec((ROWS_BLK // 128, 128), lambda i: (i, 0)),
        ],
        out_shape=[
            jax.ShapeDtypeStruct((W_ROWS // 2, 128), jnp.float32),
            jax.ShapeDtypeStruct((W_ROWS // 2, 128), jnp.float32),
        ],
    )(t3d, x_t, x_t)


def _tc_fin_body(a_ref, w0_ref, w1_ref, o_ref):
    a = a_ref[...]
    w0 = w0_ref[...]
    w1 = w1_ref[...]
    rpb = ROWS_BLK // 128
    s = jnp.float32(0.0)
    for i in range(NUM // 2 // ROWS_BLK):
        a0 = a[2 * rpb * i:2 * rpb * i + rpb, :]
        a1 = a[2 * rpb * i + rpb:2 * rpb * (i + 1), :]
        s += jnp.sum(a0 * w0[rpb * i:rpb * (i + 1), :])
        s += jnp.sum(a1 * w1[rpb * i:rpb * (i + 1), :])
    o_ref[0, 0] = s * (-1.0 / NUM)


def _tc_finish(a2d, w0, w1):
    out = pl.pallas_call(
        _tc_fin_body,
        out_shape=jax.ShapeDtypeStruct((1, 1), jnp.float32),
        out_specs=pl.BlockSpec(memory_space=pltpu.SMEM),
    )(a2d, w0, w1)
    return out[0, 0]


def kernel(inputs, targets, alpha):
    tgt = targets.astype(jnp.int32)
    tgt2d = tgt.reshape(W_ROWS, 128)
    al = jnp.pad(alpha.reshape(-1), (0, C_PAD - C))
    a2d = _sc_alpha_gather(tgt2d, al)
    w0, w1 = _tc_main(tgt, inputs)
    return _tc_finish(a2d, w0, w1)
